# baseline (speedup 1.0000x reference)
import functools

import jax
import jax.numpy as jnp
from jax import lax
from jax.experimental import pallas as pl
from jax.experimental.pallas import tpu as pltpu
from jax.experimental.pallas import tpu_sc as plsc

D_VOCAB = 100000
D_MODEL = 768
NTOK = 4 * 2048

_NC = 2
_NS = 16
_NW = _NC * _NS
_B_PER_W = NTOK // _NW
_CHUNK = 64
_NCHUNK = _B_PER_W // _CHUNK

_mesh = plsc.VectorSubcoreMesh(core_axis_name="c", subcore_axis_name="s")


@functools.partial(
    pl.kernel,
    mesh=_mesh,
    compiler_params=pltpu.CompilerParams(needs_layout_passes=False),
    out_type=jax.ShapeDtypeStruct((NTOK, D_MODEL), jnp.float32),
    scratch_types=[
        pltpu.VMEM((_NCHUNK, _CHUNK), jnp.int32),
        pltpu.VMEM((_CHUNK, D_MODEL), jnp.float32),
        pltpu.VMEM((_CHUNK, D_MODEL), jnp.float32),
        pltpu.SemaphoreType.DMA,
        pltpu.SemaphoreType.DMA,
        pltpu.SemaphoreType.DMA,
        pltpu.SemaphoreType.DMA,
        pltpu.SemaphoreType.DMA,
    ],
)
def _sc_gather(x_hbm, wt_hbm, out_hbm, idx_v, rows0, rows1,
               isem, gsem0, gsem1, wsem0, wsem1):
    wid = lax.axis_index("s") * _NC + lax.axis_index("c")
    base = wid * _B_PER_W
    rows = (rows0, rows1)
    gsem = (gsem0, gsem1)
    wsem = (wsem0, wsem1)

    icopies = [
        pltpu.async_copy(x_hbm.at[pl.ds(base + j * _CHUNK, _CHUNK)],
                         idx_v.at[j], isem)
        for j in range(_NCHUNK)
    ]
    for c in icopies:
        c.wait()

    gathers = [None] * _NCHUNK
    writes = [None] * _NCHUNK
    gathers[0] = pltpu.async_copy(wt_hbm.at[idx_v.at[0]], rows[0], gsem[0])
    gathers[1] = pltpu.async_copy(wt_hbm.at[idx_v.at[1]], rows[1], gsem[1])
    for j in range(_NCHUNK):
        b = j % 2
        gathers[j].wait()
        writes[j] = pltpu.async_copy(
            rows[b], out_hbm.at[pl.ds(base + j * _CHUNK, _CHUNK)], wsem[b])
        if j + 2 < _NCHUNK:
            writes[j].wait()
            gathers[j + 2] = pltpu.async_copy(
                wt_hbm.at[idx_v.at[j + 2]], rows[b], gsem[b])
    writes[_NCHUNK - 2].wait()
    writes[_NCHUNK - 1].wait()


def kernel(x, W_E):
    b, p = x.shape
    xf = x.reshape(-1)
    out = _sc_gather(xf, W_E.T)
    return out.reshape(b, p, D_MODEL)


# speedup vs baseline: 1.0063x; 1.0063x over previous
import functools

import jax
import jax.numpy as jnp
from jax import lax
from jax.experimental import pallas as pl
from jax.experimental.pallas import tpu as pltpu
from jax.experimental.pallas import tpu_sc as plsc

D_VOCAB = 100000
D_MODEL = 768
NTOK = 4 * 2048

_NC = 2
_NS = 16
_NW = _NC * _NS
_B_PER_W = NTOK // _NW
_CHUNK = 128
_NCHUNK = _B_PER_W // _CHUNK
_W_PER_ROW = 2048 // _B_PER_W

_mesh = plsc.VectorSubcoreMesh(core_axis_name="c", subcore_axis_name="s")


@functools.partial(
    pl.kernel,
    mesh=_mesh,
    compiler_params=pltpu.CompilerParams(needs_layout_passes=False),
    out_type=jax.ShapeDtypeStruct((NTOK, D_MODEL), jnp.float32),
    scratch_types=[
        pltpu.VMEM((_NCHUNK, _CHUNK), jnp.int32),
        pltpu.VMEM((_CHUNK, D_MODEL), jnp.float32),
        pltpu.SemaphoreType.DMA,
        pltpu.SemaphoreType.DMA,
        pltpu.SemaphoreType.DMA,
    ],
)
def _sc_gather(x_hbm, wt_hbm, out_hbm, idx_v, rows_v, isem, gsem, wsem):
    wid = lax.axis_index("s") * _NC + lax.axis_index("c")
    base = wid * _B_PER_W
    xrow = wid // _W_PER_ROW
    xcol = (wid % _W_PER_ROW) * _B_PER_W

    icopies = [
        pltpu.async_copy(x_hbm.at[xrow, pl.ds(xcol + j * _CHUNK, _CHUNK)],
                         idx_v.at[j], isem)
        for j in range(_NCHUNK)
    ]
    for c in icopies:
        c.wait()

    write = None
    for j in range(_NCHUNK):
        pltpu.async_copy(wt_hbm.at[idx_v.at[j]], rows_v, gsem).wait()
        if write is not None:
            write.wait()
        write = pltpu.async_copy(
            rows_v, out_hbm.at[pl.ds(base + j * _CHUNK, _CHUNK)], wsem)
    write.wait()


def kernel(x, W_E):
    b, p = x.shape
    out = _sc_gather(x, W_E.T)
    return out.reshape(b, p, D_MODEL)


# speedup vs baseline: 1.0141x; 1.0078x over previous
import functools

import jax
import jax.numpy as jnp
from jax import lax
from jax.experimental import pallas as pl
from jax.experimental.pallas import tpu as pltpu
from jax.experimental.pallas import tpu_sc as plsc

D_VOCAB = 100000
D_MODEL = 768
NTOK = 4 * 2048

_NC = 2
_NS = 16
_NW = _NC * _NS
_B_PER_W = NTOK // _NW
_CHUNK = 32
_NBUF = 4
_NCHUNK = _B_PER_W // _CHUNK

_mesh = plsc.VectorSubcoreMesh(core_axis_name="c", subcore_axis_name="s")


@functools.partial(
    pl.kernel,
    mesh=_mesh,
    compiler_params=pltpu.CompilerParams(needs_layout_passes=False),
    out_type=jax.ShapeDtypeStruct((NTOK, D_MODEL), jnp.float32),
    scratch_types=[
        pltpu.VMEM((_NCHUNK, _CHUNK), jnp.int32),
        pltpu.VMEM((_NBUF, _CHUNK, D_MODEL), jnp.float32),
        pltpu.SemaphoreType.DMA,
    ]
    + [pltpu.SemaphoreType.DMA] * _NBUF
    + [pltpu.SemaphoreType.DMA] * _NBUF,
)
def _sc_gather(x_hbm, wt_hbm, out_hbm, idx_v, rows_v, isem, *sems):
    gsem = sems[:_NBUF]
    wsem = sems[_NBUF:]
    wid = lax.axis_index("s") * _NC + lax.axis_index("c")
    base = wid * _B_PER_W

    icopies = [
        pltpu.async_copy(x_hbm.at[pl.ds(base + j * _CHUNK, _CHUNK)],
                         idx_v.at[j], isem)
        for j in range(_NCHUNK)
    ]
    for c in icopies:
        c.wait()

    gathers = [None] * _NCHUNK
    writes = [None] * _NCHUNK
    for j in range(_NBUF):
        gathers[j] = pltpu.async_copy(wt_hbm.at[idx_v.at[j]],
                                      rows_v.at[j % _NBUF], gsem[j % _NBUF])
    for j in range(_NCHUNK):
        b = j % _NBUF
        gathers[j].wait()
        writes[j] = pltpu.async_copy(
            rows_v.at[b], out_hbm.at[pl.ds(base + j * _CHUNK, _CHUNK)], wsem[b])
        if j + _NBUF < _NCHUNK:
            writes[j].wait()
            gathers[j + _NBUF] = pltpu.async_copy(
                wt_hbm.at[idx_v.at[j + _NBUF]], rows_v.at[b], gsem[b])
    for j in range(_NCHUNK - _NBUF, _NCHUNK):
        writes[j].wait()


def kernel(x, W_E):
    b, p = x.shape
    xf = x.reshape(-1)
    out = _sc_gather(xf, W_E.T)
    return out.reshape(b, p, D_MODEL)


# speedup vs baseline: 1.0172x; 1.0030x over previous
import functools

import jax
import jax.numpy as jnp
from jax import lax
from jax.experimental import pallas as pl
from jax.experimental.pallas import tpu as pltpu
from jax.experimental.pallas import tpu_sc as plsc

D_VOCAB = 100000
D_MODEL = 768
NTOK = 4 * 2048

_NC = 2
_NS = 16
_NW = _NC * _NS
_B_PER_W = NTOK // _NW
_CHUNK = 32
_NBUF = 4
_NCHUNK = _B_PER_W // _CHUNK

_mesh = plsc.VectorSubcoreMesh(core_axis_name="c", subcore_axis_name="s")


@functools.partial(
    pl.kernel,
    mesh=_mesh,
    compiler_params=pltpu.CompilerParams(
        needs_layout_passes=False,
        skip_device_barrier=True,
        disable_bounds_checks=True,
        disable_semaphore_checks=True,
    ),
    out_type=jax.ShapeDtypeStruct((NTOK, D_MODEL), jnp.float32),
    scratch_types=[
        pltpu.VMEM((_NCHUNK, _CHUNK), jnp.int32),
        pltpu.VMEM((_NBUF, _CHUNK, D_MODEL), jnp.float32),
        pltpu.SemaphoreType.DMA,
    ]
    + [pltpu.SemaphoreType.DMA] * _NBUF
    + [pltpu.SemaphoreType.DMA] * _NBUF,
)
def _sc_gather(x_hbm, wt_hbm, out_hbm, idx_v, rows_v, isem, *sems):
    gsem = sems[:_NBUF]
    wsem = sems[_NBUF:]
    wid = lax.axis_index("s") * _NC + lax.axis_index("c")
    base = wid * _B_PER_W

    icopies = [
        pltpu.async_copy(x_hbm.at[pl.ds(base + j * _CHUNK, _CHUNK)],
                         idx_v.at[j], isem)
        for j in range(_NCHUNK)
    ]
    for c in icopies:
        c.wait()

    gathers = [None] * _NCHUNK
    writes = [None] * _NCHUNK
    for j in range(_NBUF):
        gathers[j] = pltpu.async_copy(wt_hbm.at[idx_v.at[j]],
                                      rows_v.at[j % _NBUF], gsem[j % _NBUF])
    for j in range(_NCHUNK):
        b = j % _NBUF
        gathers[j].wait()
        writes[j] = pltpu.async_copy(
            rows_v.at[b], out_hbm.at[pl.ds(base + j * _CHUNK, _CHUNK)], wsem[b])
        if j + _NBUF < _NCHUNK:
            writes[j].wait()
            gathers[j + _NBUF] = pltpu.async_copy(
                wt_hbm.at[idx_v.at[j + _NBUF]], rows_v.at[b], gsem[b])
    for j in range(_NCHUNK - _NBUF, _NCHUNK):
        writes[j].wait()


def kernel(x, W_E):
    b, p = x.shape
    xf = x.reshape(-1)
    out = _sc_gather(xf, W_E.T)
    return out.reshape(b, p, D_MODEL)
